# Initial kernel scaffold; baseline (speedup 1.0000x reference)
import jax
import jax.numpy as jnp
from jax.experimental import pallas as pl


def kernel(cis, dis, edge_index, edge_attr, W_r, b_r, W_d, b_d, qkv_W, proj_W, proj_b, ln1_g, ln1_b, lin1_W, lin1_b, lin2_W, lin2_b, ln2_g, ln2_b, out_W, out_b, pe):
    raise NotImplementedError("write your pallas kernel here")


# speedup vs baseline: 4.5303x; 4.5303x over previous
import functools

import jax
import jax.numpy as jnp
from jax import lax
from jax.experimental import pallas as pl
from jax.experimental.pallas import tpu as pltpu
from jax.experimental.pallas import tpu_sc as plsc

_D = 256
_DH = 128
_DFF = 1024
_N = 10000
_NPAD = 10240
_NHALF = 5008
_ACC_ROWS = 5120
_E = 160000
_EPAD = 163840
_CHUNK = _EPAD // 16
_B = 80
_NB = _CHUNK // _B
_DEG_B = 128
_DEG_NB = _CHUNK // _DEG_B
_BLK = 1000
_NBLK = _N // _BLK


def _graph_sc_body(slo_hbm, shi_hbm, row_hbm, col_hbm, w_hbm,
                   olo_hbm, ohi_hbm, dinv_hbm,
                   rowv, colv, wv, nwv, dinv_v,
                   st_row, st_col, st_di, st_dw, zb, buf0, buf1,
                   acc, dacc,
                   sem0, sem1):
    c = lax.axis_index("c")
    s = lax.axis_index("s")
    base_e = s * _CHUNK

    pltpu.sync_copy(row_hbm.at[pl.ds(base_e, _CHUNK)], rowv)
    pltpu.sync_copy(col_hbm.at[pl.ds(base_e, _CHUNK)], colv)
    pltpu.sync_copy(w_hbm.at[pl.ds(base_e, _CHUNK)], wv)

    zero16 = jnp.zeros((16,), jnp.float32)

    def _zd(i, carry):
        zb[pl.ds(i * 16, 16)] = zero16
        return carry

    lax.fori_loop(0, (_NPAD // 16) // 16, _zd, 0)
    pltpu.sync_copy(zb, dacc.at[pl.ds(s * (_NPAD // 16), _NPAD // 16)])
    plsc.subcore_barrier()

    def _deg(j, carry):
        off = j * _DEG_B
        for v2 in range(_DEG_B // 16):
            st_di[0, pl.ds(v2 * 16, 16)] = rowv[pl.ds(off + v2 * 16, 16)]
            st_dw[0, pl.ds(v2 * 16, 16)] = wv[pl.ds(off + v2 * 16, 16)]
        pltpu.sync_copy(st_dw.at[0], dacc.at[st_di.at[0]], add=True)
        return carry

    lax.fori_loop(0, _DEG_NB, _deg, 0)
    plsc.subcore_barrier()

    pltpu.sync_copy(dacc, dinv_v)
    magic = jnp.full((16,), 0x5F3759DF, jnp.int32)

    def _rs(i, carry):
        x = dinv_v[pl.ds(i * 16, 16)] + 1.0
        bits = lax.bitcast_convert_type(x, jnp.int32)
        y = lax.bitcast_convert_type(magic - lax.shift_right_arithmetic(bits, 1),
                                     jnp.float32)
        hx = 0.5 * x
        for _ in range(3):
            y = y * (1.5 - hx * y * y)
        dinv_v[pl.ds(i * 16, 16)] = y
        return carry

    lax.fori_loop(0, _NPAD // 16, _rs, 0)

    @pl.when(jnp.logical_and(c == 0, s == 0))
    def _():
        pltpu.sync_copy(dinv_v, dinv_hbm)

    def _stage(i, slot):
        off = i * _B
        for v2 in range(_B // 16):
            r16 = rowv[pl.ds(off + v2 * 16, 16)]
            c16 = colv[pl.ds(off + v2 * 16, 16)]
            w16 = wv[pl.ds(off + v2 * 16, 16)]
            loc = r16 - c * _NHALF
            ok = jnp.logical_and(loc >= 0, loc < _NHALF)
            loc = jnp.where(ok, loc, _NHALF + s)
            st_row[slot, pl.ds(v2 * 16, 16)] = loc
            st_col[slot, pl.ds(v2 * 16, 16)] = c16
            dr = plsc.load_gather(dinv_v, [r16])
            dc = plsc.load_gather(dinv_v, [c16])
            nwv[pl.ds(slot * _B + v2 * 16, 16)] = dr * w16 * dc

    def _edge_pass(src_hbm, out_hbm):
        def _zrow(e, carry):
            for v2 in range(_DH // 16):
                buf0[e, pl.ds(v2 * 16, 16)] = zero16
            return carry

        lax.fori_loop(0, _B, _zrow, 0)
        for t in range(_ACC_ROWS // 16 // _B):
            pltpu.sync_copy(buf0,
                            acc.at[pl.ds(s * (_ACC_ROWS // 16) + t * _B, _B)])
        plsc.subcore_barrier()

        def _start(slot, buf, sem):
            pltpu.make_async_copy(src_hbm.at[st_col.at[slot]], buf, sem).start()

        def _process(slot, buf, sem):
            pltpu.make_async_copy(src_hbm.at[st_col.at[slot]], buf, sem).wait()

            def _scale(e, carry):
                sp = plsc.load_gather(
                    nwv, [jnp.broadcast_to(slot * _B + e, (16,))])
                for v2 in range(_DH // 16):
                    buf[e, pl.ds(v2 * 16, 16)] = buf[e, pl.ds(v2 * 16, 16)] * sp
                return carry

            lax.fori_loop(0, _B, _scale, 0)
            pltpu.sync_copy(buf, acc.at[st_row.at[slot]], add=True)

        _stage(0, 0)
        _start(0, buf0, sem0)

        def _outer(jj, carry):
            j = jj * 2
            _stage(j + 1, 1)
            _start(1, buf1, sem1)
            _process(0, buf0, sem0)

            @pl.when(j + 2 < _NB)
            def _():
                _stage(j + 2, 0)
                _start(0, buf0, sem0)

            _process(1, buf1, sem1)
            return carry

        lax.fori_loop(0, _NB // 2, _outer, 0)
        plsc.subcore_barrier()

        @pl.when(s < 15)
        def _():
            pltpu.sync_copy(acc.at[pl.ds(s * 320, 320)],
                            out_hbm.at[pl.ds(c * _NHALF + s * 320, 320)])

        @pl.when(s == 15)
        def _():
            pltpu.sync_copy(acc.at[pl.ds(4800, _NHALF - 4800)],
                            out_hbm.at[pl.ds(c * _NHALF + 4800, _NHALF - 4800)])

        plsc.subcore_barrier()

    _edge_pass(slo_hbm, olo_hbm)
    _edge_pass(shi_hbm, ohi_hbm)


@functools.cache
def _make_graph_sc():
    return pl.kernel(
        _graph_sc_body,
        out_type=[
            jax.ShapeDtypeStruct((2 * _NHALF, _DH), jnp.float32),
            jax.ShapeDtypeStruct((2 * _NHALF, _DH), jnp.float32),
            jax.ShapeDtypeStruct((_NPAD,), jnp.float32),
        ],
        mesh=plsc.VectorSubcoreMesh(core_axis_name="c", subcore_axis_name="s",
                                    num_cores=2, num_subcores=16),
        compiler_params=pltpu.CompilerParams(use_tc_tiling_on_sc=False,
                                             needs_layout_passes=False),
        scratch_types=[
            pltpu.VMEM((_CHUNK,), jnp.int32),
            pltpu.VMEM((_CHUNK,), jnp.int32),
            pltpu.VMEM((_CHUNK,), jnp.float32),
            pltpu.VMEM((2 * _B,), jnp.float32),
            pltpu.VMEM((_NPAD,), jnp.float32),
            pltpu.VMEM((2, _B), jnp.int32),
            pltpu.VMEM((2, _B), jnp.int32),
            pltpu.VMEM((1, _DEG_B), jnp.int32),
            pltpu.VMEM((1, _DEG_B), jnp.float32),
            pltpu.VMEM(((_NPAD // 16),), jnp.float32),
            pltpu.VMEM((_B, _DH), jnp.float32),
            pltpu.VMEM((_B, _DH), jnp.float32),
            pltpu.VMEM_SHARED((_ACC_ROWS, _DH), jnp.float32),
            pltpu.VMEM_SHARED((_NPAD,), jnp.float32),
            pltpu.SemaphoreType.DMA,
            pltpu.SemaphoreType.DMA,
        ],
    )


def _ln(x, g, b):
    mu = jnp.mean(x, axis=-1, keepdims=True)
    var = jnp.mean((x - mu) ** 2, axis=-1, keepdims=True)
    return (x - mu) * lax.rsqrt(var + 1e-5) * g + b


def _tc1_body(cis_ref, dis_ref, wr_ref, br_ref, wd_ref, bd_ref, qkvw_ref,
              pe_ref, x_ref, q_ref, kv_ref, ksum_ref):
    pid = pl.program_id(0)
    cblk = jnp.dot(cis_ref[...], wr_ref[...],
                   preferred_element_type=jnp.float32) + br_ref[...]
    dblk = jnp.dot(dis_ref[...], wd_ref[...],
                   preferred_element_type=jnp.float32) + bd_ref[...]
    x = jnp.where(pid < _NBLK // 2, cblk, dblk) + pe_ref[...]
    x_ref[...] = x
    qkv = jnp.dot(x, qkvw_ref[...], preferred_element_type=jnp.float32)
    q = jnp.maximum(qkv[:, :_D], 0.0)
    k = jnp.maximum(qkv[:, _D:2 * _D], 0.0)
    v = qkv[:, 2 * _D:]
    q_ref[...] = q
    kvb = lax.dot_general(k, v, (((0,), (0,)), ((), ())),
                          preferred_element_type=jnp.float32)
    ksb = jnp.broadcast_to(jnp.sum(k, axis=0, keepdims=True), (8, _D))

    @pl.when(pid == 0)
    def _():
        kv_ref[...] = kvb
        ksum_ref[...] = ksb

    @pl.when(pid > 0)
    def _():
        kv_ref[...] += kvb
        ksum_ref[...] += ksb


_TC1_ARGS = dict(
    grid=(_NBLK,),
    in_specs=[
        pl.BlockSpec((_BLK, 128), lambda i: (jnp.minimum(i, _NBLK // 2 - 1), 0)),
        pl.BlockSpec((_BLK, 128), lambda i: (jnp.maximum(i - _NBLK // 2, 0), 0)),
        pl.BlockSpec((128, _D), lambda i: (0, 0)),
        pl.BlockSpec((1, _D), lambda i: (0, 0)),
        pl.BlockSpec((128, _D), lambda i: (0, 0)),
        pl.BlockSpec((1, _D), lambda i: (0, 0)),
        pl.BlockSpec((_D, 3 * _D), lambda i: (0, 0)),
        pl.BlockSpec((_BLK, _D), lambda i: (i, 0)),
    ],
    out_specs=[
        pl.BlockSpec((_BLK, _D), lambda i: (i, 0)),
        pl.BlockSpec((_BLK, _D), lambda i: (i, 0)),
        pl.BlockSpec((_D, _D), lambda i: (0, 0)),
        pl.BlockSpec((8, _D), lambda i: (0, 0)),
    ],
    out_shape=[
        jax.ShapeDtypeStruct((_N, _D), jnp.float32),
        jax.ShapeDtypeStruct((_N, _D), jnp.float32),
        jax.ShapeDtypeStruct((_D, _D), jnp.float32),
        jax.ShapeDtypeStruct((8, _D), jnp.float32),
    ],
)
_tc1 = pl.pallas_call(_tc1_body, **_TC1_ARGS)


def _tc2_body(x_ref, q_ref, kv_ref, ksum_ref, pw_ref, pb_ref, g1_ref, b1_ref,
              l1w_ref, l1b_ref, l2w_ref, l2b_ref, g2_ref, b2_ref, src_ref,
              slo_ref, shi_ref):
    x = x_ref[...]
    q = q_ref[...]
    denom = jnp.sum(q * ksum_ref[0:1, :], axis=-1, keepdims=True) + 1e-6
    att = jnp.dot(q, kv_ref[...], preferred_element_type=jnp.float32) / denom
    t = jnp.dot(att, pw_ref[...], preferred_element_type=jnp.float32) + pb_ref[...]
    s1 = x + _ln(t, g1_ref[...], b1_ref[...])
    h = jnp.maximum(
        jnp.dot(s1, l1w_ref[...], preferred_element_type=jnp.float32)
        + l1b_ref[...], 0.0)
    s2 = jnp.dot(h, l2w_ref[...], preferred_element_type=jnp.float32) + l2b_ref[...]
    src = s1 + _ln(s2, g2_ref[...], b2_ref[...])
    src_ref[...] = src
    slo_ref[...] = src[:, :_DH]
    shi_ref[...] = src[:, _DH:]


_TC2_ARGS = dict(
    grid=(_NBLK,),
    in_specs=[
        pl.BlockSpec((_BLK, _D), lambda i: (i, 0)),
        pl.BlockSpec((_BLK, _D), lambda i: (i, 0)),
        pl.BlockSpec((_D, _D), lambda i: (0, 0)),
        pl.BlockSpec((8, _D), lambda i: (0, 0)),
        pl.BlockSpec((_D, _D), lambda i: (0, 0)),
        pl.BlockSpec((1, _D), lambda i: (0, 0)),
        pl.BlockSpec((1, _D), lambda i: (0, 0)),
        pl.BlockSpec((1, _D), lambda i: (0, 0)),
        pl.BlockSpec((_D, _DFF), lambda i: (0, 0)),
        pl.BlockSpec((1, _DFF), lambda i: (0, 0)),
        pl.BlockSpec((_DFF, _D), lambda i: (0, 0)),
        pl.BlockSpec((1, _D), lambda i: (0, 0)),
        pl.BlockSpec((1, _D), lambda i: (0, 0)),
        pl.BlockSpec((1, _D), lambda i: (0, 0)),
    ],
    out_specs=[
        pl.BlockSpec((_BLK, _D), lambda i: (i, 0)),
        pl.BlockSpec((_BLK, _DH), lambda i: (i, 0)),
        pl.BlockSpec((_BLK, _DH), lambda i: (i, 0)),
    ],
    out_shape=[
        jax.ShapeDtypeStruct((_N, _D), jnp.float32),
        jax.ShapeDtypeStruct((_N, _DH), jnp.float32),
        jax.ShapeDtypeStruct((_N, _DH), jnp.float32),
    ],
)
_tc2 = pl.pallas_call(_tc2_body, **_TC2_ARGS)


def _tc3_body(lo_ref, hi_ref, src_ref, dinv_ref, ow_ref, ob_ref, o_ref):
    dv = dinv_ref[...]
    scat = jnp.concatenate([lo_ref[...], hi_ref[...]], axis=-1)
    g = jnp.maximum(scat + src_ref[...] * (dv * dv), 0.0)
    o_ref[...] = jnp.dot(g, ow_ref[...],
                         preferred_element_type=jnp.float32) + ob_ref[...]


_TC3_ARGS = dict(
    grid=(_NBLK,),
    in_specs=[
        pl.BlockSpec((_BLK, _DH), lambda i: (i, 0)),
        pl.BlockSpec((_BLK, _DH), lambda i: (i, 0)),
        pl.BlockSpec((_BLK, _D), lambda i: (i, 0)),
        pl.BlockSpec((_BLK, 1), lambda i: (i, 0)),
        pl.BlockSpec((_D, _D), lambda i: (0, 0)),
        pl.BlockSpec((1, _D), lambda i: (0, 0)),
    ],
    out_specs=pl.BlockSpec((_BLK, _D), lambda i: (i, 0)),
    out_shape=jax.ShapeDtypeStruct((_N, _D), jnp.float32),
)
_tc3 = pl.pallas_call(_tc3_body, **_TC3_ARGS)


def kernel(cis, dis, edge_index, edge_attr, W_r, b_r, W_d, b_d, qkv_W, proj_W,
           proj_b, ln1_g, ln1_b, lin1_W, lin1_b, lin2_W, lin2_b, ln2_g, ln2_b,
           out_W, out_b, pe):
    row = jnp.pad(edge_index[0], (0, _EPAD - _E))
    col = jnp.pad(edge_index[1], (0, _EPAD - _E))
    w = jnp.pad(edge_attr, (0, _EPAD - _E))

    x, q, kv, ksum = _tc1(cis, dis, W_r, b_r.reshape(1, _D), W_d,
                          b_d.reshape(1, _D), qkv_W, pe)
    src, slo, shi = _tc2(x, q, kv, ksum, proj_W, proj_b.reshape(1, _D),
                         ln1_g.reshape(1, _D), ln1_b.reshape(1, _D), lin1_W,
                         lin1_b.reshape(1, _DFF), lin2_W, lin2_b.reshape(1, _D),
                         ln2_g.reshape(1, _D), ln2_b.reshape(1, _D))
    olo, ohi, dinv = _make_graph_sc()(slo, shi, row, col, w)
    return _tc3(olo[:_N], ohi[:_N], src, dinv[:_N].reshape(_N, 1), out_W,
                out_b.reshape(1, _D))


# speedup vs baseline: 5.1334x; 1.1331x over previous
import functools

import jax
import jax.numpy as jnp
from jax import lax
from jax.experimental import pallas as pl
from jax.experimental.pallas import tpu as pltpu
from jax.experimental.pallas import tpu_sc as plsc

_D = 256
_DH = 128
_DFF = 1024
_N = 10000
_NPAD = 10240
_NHALF = 5008
_ACC_ROWS = 5120
_E = 160000
_EPAD = 163840
_CHUNK = _EPAD // 16
_B = 80
_NB = _CHUNK // _B
_DEG_B = 128
_DEG_NB = _CHUNK // _DEG_B
_BLK = 1000
_NBLK = _N // _BLK


def _graph_sc_body(slo_hbm, shi_hbm, row_hbm, col_hbm, w_hbm,
                   olo_hbm, ohi_hbm, dinv_hbm,
                   rowv, colv, wv, nwv, dinv_v,
                   st_row, st_col, st_di, st_dw, zb,
                   gbuf0, gbuf1, tbuf0, tbuf1,
                   acc, dacc,
                   gsem0, gsem1, ssem0, ssem1):
    c = lax.axis_index("c")
    s = lax.axis_index("s")
    base_e = s * _CHUNK

    pltpu.sync_copy(row_hbm.at[pl.ds(base_e, _CHUNK)], rowv)
    pltpu.sync_copy(col_hbm.at[pl.ds(base_e, _CHUNK)], colv)
    pltpu.sync_copy(w_hbm.at[pl.ds(base_e, _CHUNK)], wv)

    zero16 = jnp.zeros((16,), jnp.float32)

    def _zd(i, carry):
        zb[pl.ds(i * 16, 16)] = zero16
        return carry

    lax.fori_loop(0, (_NPAD // 16) // 16, _zd, 0)
    pltpu.sync_copy(zb, dacc.at[pl.ds(s * (_NPAD // 16), _NPAD // 16)])
    plsc.subcore_barrier()

    def _deg(j, carry):
        off = j * _DEG_B
        for v2 in range(_DEG_B // 16):
            st_di[0, pl.ds(v2 * 16, 16)] = rowv[pl.ds(off + v2 * 16, 16)]
            st_dw[0, pl.ds(v2 * 16, 16)] = wv[pl.ds(off + v2 * 16, 16)]
        pltpu.sync_copy(st_dw.at[0], dacc.at[st_di.at[0]], add=True)
        return carry

    lax.fori_loop(0, _DEG_NB, _deg, 0)
    plsc.subcore_barrier()

    pltpu.sync_copy(dacc, dinv_v)
    magic = jnp.full((16,), 0x5F3759DF, jnp.int32)

    def _rs(i, carry):
        x = dinv_v[pl.ds(i * 16, 16)] + 1.0
        bits = lax.bitcast_convert_type(x, jnp.int32)
        y = lax.bitcast_convert_type(magic - lax.shift_right_arithmetic(bits, 1),
                                     jnp.float32)
        hx = 0.5 * x
        for _ in range(3):
            y = y * (1.5 - hx * y * y)
        dinv_v[pl.ds(i * 16, 16)] = y
        return carry

    lax.fori_loop(0, _NPAD // 16, _rs, 0)

    @pl.when(jnp.logical_and(c == 0, s == 0))
    def _():
        pltpu.sync_copy(dinv_v, dinv_hbm)

    def _stage(i, slot):
        off = i * _B
        for v2 in range(_B // 16):
            r16 = rowv[pl.ds(off + v2 * 16, 16)]
            c16 = colv[pl.ds(off + v2 * 16, 16)]
            w16 = wv[pl.ds(off + v2 * 16, 16)]
            loc = r16 - c * _NHALF
            ok = jnp.logical_and(loc >= 0, loc < _NHALF)
            loc = jnp.where(ok, loc, _NHALF + s)
            st_row[slot, pl.ds(v2 * 16, 16)] = loc
            st_col[slot, pl.ds(v2 * 16, 16)] = c16
            dr = plsc.load_gather(dinv_v, [r16])
            dc = plsc.load_gather(dinv_v, [c16])
            nwv[pl.ds(slot * _B + v2 * 16, 16)] = dr * w16 * dc

    gbufs = (gbuf0, gbuf1)
    tbufs = (tbuf0, tbuf1)
    gsems = (gsem0, gsem1)
    ssems = (ssem0, ssem1)

    def _edge_pass(src_hbm, out_hbm):
        def _zrow(e, carry):
            for v2 in range(_DH // 16):
                gbuf0[e, pl.ds(v2 * 16, 16)] = zero16
            return carry

        lax.fori_loop(0, _B, _zrow, 0)
        for t in range(_ACC_ROWS // 16 // _B):
            pltpu.sync_copy(gbuf0,
                            acc.at[pl.ds(s * (_ACC_ROWS // 16) + t * _B, _B)])
        plsc.subcore_barrier()

        def _gstart(slot, k2):
            pltpu.make_async_copy(src_hbm.at[st_col.at[slot]], gbufs[k2],
                                  gsems[k2]).start()

        _stage(0, 0)
        _gstart(0, 0)
        _stage(1, 1)
        _gstart(1, 1)

        def _outer(jj, carry):
            j = jj * 4
            for b in range(4):
                i = j + b
                k2 = b % 2

                @pl.when(i >= 2)
                def _():
                    pltpu.make_async_copy(
                        tbufs[k2], acc.at[st_row.at[b]], ssems[k2]).wait()

                pltpu.make_async_copy(src_hbm.at[st_col.at[b]], gbufs[k2],
                                      gsems[k2]).wait()

                g, tb = gbufs[k2], tbufs[k2]

                @plsc.parallel_loop(0, _B, 1, unroll=4)
                def _scale(e):
                    sp = plsc.load_gather(
                        nwv, [jnp.broadcast_to(b * _B + e, (16,))])
                    for v2 in range(_DH // 16):
                        tb[e, pl.ds(v2 * 16, 16)] = g[e, pl.ds(v2 * 16, 16)] * sp

                pltpu.async_copy(tb, acc.at[st_row.at[b]], ssems[k2],
                                 add=True)

                @pl.when(i + 2 < _NB)
                def _():
                    _stage(i + 2, (b + 2) % 4)
                    _gstart((b + 2) % 4, k2)

            return carry

        lax.fori_loop(0, _NB // 4, _outer, 0)
        pltpu.make_async_copy(tbufs[0], acc.at[st_row.at[0]], ssems[0]).wait()
        pltpu.make_async_copy(tbufs[1], acc.at[st_row.at[1]], ssems[1]).wait()
        plsc.subcore_barrier()

        @pl.when(s < 15)
        def _():
            pltpu.sync_copy(acc.at[pl.ds(s * 320, 320)],
                            out_hbm.at[pl.ds(c * _NHALF + s * 320, 320)])

        @pl.when(s == 15)
        def _():
            pltpu.sync_copy(acc.at[pl.ds(4800, _NHALF - 4800)],
                            out_hbm.at[pl.ds(c * _NHALF + 4800, _NHALF - 4800)])

        plsc.subcore_barrier()

    _edge_pass(slo_hbm, olo_hbm)
    _edge_pass(shi_hbm, ohi_hbm)


@functools.cache
def _make_graph_sc():
    return pl.kernel(
        _graph_sc_body,
        out_type=[
            jax.ShapeDtypeStruct((2 * _NHALF, _DH), jnp.float32),
            jax.ShapeDtypeStruct((2 * _NHALF, _DH), jnp.float32),
            jax.ShapeDtypeStruct((_NPAD,), jnp.float32),
        ],
        mesh=plsc.VectorSubcoreMesh(core_axis_name="c", subcore_axis_name="s",
                                    num_cores=2, num_subcores=16),
        compiler_params=pltpu.CompilerParams(use_tc_tiling_on_sc=False,
                                             needs_layout_passes=False),
        scratch_types=[
            pltpu.VMEM((_CHUNK,), jnp.int32),
            pltpu.VMEM((_CHUNK,), jnp.int32),
            pltpu.VMEM((_CHUNK,), jnp.float32),
            pltpu.VMEM((4 * _B,), jnp.float32),
            pltpu.VMEM((_NPAD,), jnp.float32),
            pltpu.VMEM((4, _B), jnp.int32),
            pltpu.VMEM((4, _B), jnp.int32),
            pltpu.VMEM((1, _DEG_B), jnp.int32),
            pltpu.VMEM((1, _DEG_B), jnp.float32),
            pltpu.VMEM(((_NPAD // 16),), jnp.float32),
            pltpu.VMEM((_B, _DH), jnp.float32),
            pltpu.VMEM((_B, _DH), jnp.float32),
            pltpu.VMEM((_B, _DH), jnp.float32),
            pltpu.VMEM((_B, _DH), jnp.float32),
            pltpu.VMEM_SHARED((_ACC_ROWS, _DH), jnp.float32),
            pltpu.VMEM_SHARED((_NPAD,), jnp.float32),
            pltpu.SemaphoreType.DMA,
            pltpu.SemaphoreType.DMA,
            pltpu.SemaphoreType.DMA,
            pltpu.SemaphoreType.DMA,
        ],
    )


def _ln(x, g, b):
    mu = jnp.mean(x, axis=-1, keepdims=True)
    var = jnp.mean((x - mu) ** 2, axis=-1, keepdims=True)
    return (x - mu) * lax.rsqrt(var + 1e-5) * g + b


def _tc1_body(cis_ref, dis_ref, wr_ref, br_ref, wd_ref, bd_ref, qkvw_ref,
              pe_ref, x_ref, q_ref, kv_ref, ksum_ref):
    pid = pl.program_id(0)
    cblk = jnp.dot(cis_ref[...], wr_ref[...],
                   preferred_element_type=jnp.float32) + br_ref[...]
    dblk = jnp.dot(dis_ref[...], wd_ref[...],
                   preferred_element_type=jnp.float32) + bd_ref[...]
    x = jnp.where(pid < _NBLK // 2, cblk, dblk) + pe_ref[...]
    x_ref[...] = x
    qkv = jnp.dot(x, qkvw_ref[...], preferred_element_type=jnp.float32)
    q = jnp.maximum(qkv[:, :_D], 0.0)
    k = jnp.maximum(qkv[:, _D:2 * _D], 0.0)
    v = qkv[:, 2 * _D:]
    q_ref[...] = q
    kvb = lax.dot_general(k, v, (((0,), (0,)), ((), ())),
                          preferred_element_type=jnp.float32)
    ksb = jnp.broadcast_to(jnp.sum(k, axis=0, keepdims=True), (8, _D))

    @pl.when(pid == 0)
    def _():
        kv_ref[...] = kvb
        ksum_ref[...] = ksb

    @pl.when(pid > 0)
    def _():
        kv_ref[...] += kvb
        ksum_ref[...] += ksb


_TC1_ARGS = dict(
    grid=(_NBLK,),
    in_specs=[
        pl.BlockSpec((_BLK, 128), lambda i: (jnp.minimum(i, _NBLK // 2 - 1), 0)),
        pl.BlockSpec((_BLK, 128), lambda i: (jnp.maximum(i - _NBLK // 2, 0), 0)),
        pl.BlockSpec((128, _D), lambda i: (0, 0)),
        pl.BlockSpec((1, _D), lambda i: (0, 0)),
        pl.BlockSpec((128, _D), lambda i: (0, 0)),
        pl.BlockSpec((1, _D), lambda i: (0, 0)),
        pl.BlockSpec((_D, 3 * _D), lambda i: (0, 0)),
        pl.BlockSpec((_BLK, _D), lambda i: (i, 0)),
    ],
    out_specs=[
        pl.BlockSpec((_BLK, _D), lambda i: (i, 0)),
        pl.BlockSpec((_BLK, _D), lambda i: (i, 0)),
        pl.BlockSpec((_D, _D), lambda i: (0, 0)),
        pl.BlockSpec((8, _D), lambda i: (0, 0)),
    ],
    out_shape=[
        jax.ShapeDtypeStruct((_N, _D), jnp.float32),
        jax.ShapeDtypeStruct((_N, _D), jnp.float32),
        jax.ShapeDtypeStruct((_D, _D), jnp.float32),
        jax.ShapeDtypeStruct((8, _D), jnp.float32),
    ],
)
_tc1 = pl.pallas_call(_tc1_body, **_TC1_ARGS)


def _tc2_body(x_ref, q_ref, kv_ref, ksum_ref, pw_ref, pb_ref, g1_ref, b1_ref,
              l1w_ref, l1b_ref, l2w_ref, l2b_ref, g2_ref, b2_ref, src_ref,
              slo_ref, shi_ref):
    x = x_ref[...]
    q = q_ref[...]
    denom = jnp.sum(q * ksum_ref[0:1, :], axis=-1, keepdims=True) + 1e-6
    att = jnp.dot(q, kv_ref[...], preferred_element_type=jnp.float32) / denom
    t = jnp.dot(att, pw_ref[...], preferred_element_type=jnp.float32) + pb_ref[...]
    s1 = x + _ln(t, g1_ref[...], b1_ref[...])
    h = jnp.maximum(
        jnp.dot(s1, l1w_ref[...], preferred_element_type=jnp.float32)
        + l1b_ref[...], 0.0)
    s2 = jnp.dot(h, l2w_ref[...], preferred_element_type=jnp.float32) + l2b_ref[...]
    src = s1 + _ln(s2, g2_ref[...], b2_ref[...])
    src_ref[...] = src
    slo_ref[...] = src[:, :_DH]
    shi_ref[...] = src[:, _DH:]


_TC2_ARGS = dict(
    grid=(_NBLK,),
    in_specs=[
        pl.BlockSpec((_BLK, _D), lambda i: (i, 0)),
        pl.BlockSpec((_BLK, _D), lambda i: (i, 0)),
        pl.BlockSpec((_D, _D), lambda i: (0, 0)),
        pl.BlockSpec((8, _D), lambda i: (0, 0)),
        pl.BlockSpec((_D, _D), lambda i: (0, 0)),
        pl.BlockSpec((1, _D), lambda i: (0, 0)),
        pl.BlockSpec((1, _D), lambda i: (0, 0)),
        pl.BlockSpec((1, _D), lambda i: (0, 0)),
        pl.BlockSpec((_D, _DFF), lambda i: (0, 0)),
        pl.BlockSpec((1, _DFF), lambda i: (0, 0)),
        pl.BlockSpec((_DFF, _D), lambda i: (0, 0)),
        pl.BlockSpec((1, _D), lambda i: (0, 0)),
        pl.BlockSpec((1, _D), lambda i: (0, 0)),
        pl.BlockSpec((1, _D), lambda i: (0, 0)),
    ],
    out_specs=[
        pl.BlockSpec((_BLK, _D), lambda i: (i, 0)),
        pl.BlockSpec((_BLK, _DH), lambda i: (i, 0)),
        pl.BlockSpec((_BLK, _DH), lambda i: (i, 0)),
    ],
    out_shape=[
        jax.ShapeDtypeStruct((_N, _D), jnp.float32),
        jax.ShapeDtypeStruct((_N, _DH), jnp.float32),
        jax.ShapeDtypeStruct((_N, _DH), jnp.float32),
    ],
)
_tc2 = pl.pallas_call(_tc2_body, **_TC2_ARGS)


def _tc3_body(lo_ref, hi_ref, src_ref, dinv_ref, ow_ref, ob_ref, o_ref):
    dv = dinv_ref[...]
    scat = jnp.concatenate([lo_ref[...], hi_ref[...]], axis=-1)
    g = jnp.maximum(scat + src_ref[...] * (dv * dv), 0.0)
    o_ref[...] = jnp.dot(g, ow_ref[...],
                         preferred_element_type=jnp.float32) + ob_ref[...]


_TC3_ARGS = dict(
    grid=(_NBLK,),
    in_specs=[
        pl.BlockSpec((_BLK, _DH), lambda i: (i, 0)),
        pl.BlockSpec((_BLK, _DH), lambda i: (i, 0)),
        pl.BlockSpec((_BLK, _D), lambda i: (i, 0)),
        pl.BlockSpec((_BLK, 1), lambda i: (i, 0)),
        pl.BlockSpec((_D, _D), lambda i: (0, 0)),
        pl.BlockSpec((1, _D), lambda i: (0, 0)),
    ],
    out_specs=pl.BlockSpec((_BLK, _D), lambda i: (i, 0)),
    out_shape=jax.ShapeDtypeStruct((_N, _D), jnp.float32),
)
_tc3 = pl.pallas_call(_tc3_body, **_TC3_ARGS)


def kernel(cis, dis, edge_index, edge_attr, W_r, b_r, W_d, b_d, qkv_W, proj_W,
           proj_b, ln1_g, ln1_b, lin1_W, lin1_b, lin2_W, lin2_b, ln2_g, ln2_b,
           out_W, out_b, pe):
    row = jnp.pad(edge_index[0], (0, _EPAD - _E))
    col = jnp.pad(edge_index[1], (0, _EPAD - _E))
    w = jnp.pad(edge_attr, (0, _EPAD - _E))

    x, q, kv, ksum = _tc1(cis, dis, W_r, b_r.reshape(1, _D), W_d,
                          b_d.reshape(1, _D), qkv_W, pe)
    src, slo, shi = _tc2(x, q, kv, ksum, proj_W, proj_b.reshape(1, _D),
                         ln1_g.reshape(1, _D), ln1_b.reshape(1, _D), lin1_W,
                         lin1_b.reshape(1, _DFF), lin2_W, lin2_b.reshape(1, _D),
                         ln2_g.reshape(1, _D), ln2_b.reshape(1, _D))
    olo, ohi, dinv = _make_graph_sc()(slo, shi, row, col, w)
    return _tc3(olo[:_N], ohi[:_N], src, dinv[:_N].reshape(_N, 1), out_W,
                out_b.reshape(1, _D))
